# baseline (speedup 1.0000x reference)
import jax
import jax.numpy as jnp
from jax import lax
from jax.experimental import pallas as pl
from jax.experimental.pallas import tpu as pltpu
from jax.experimental.pallas import tpu_sc as plsc

MAXLEN = 200
EMBED_DIM = 64
CHUNK = 128
NUM_CORES = 2
NUM_SUBCORES = 16
NUM_WORKERS = NUM_CORES * NUM_SUBCORES


def _sc_body(x_hbm, pos2_hbm, table_hbm, out_hbm,
             idx_v, pos_v, rows_v, sem0, sem1):
    c = lax.axis_index("c")
    s = lax.axis_index("s")
    wid = s * NUM_CORES + c
    chunks_per_worker = x_hbm.shape[0] // NUM_WORKERS

    pltpu.sync_copy(x_hbm.at[pl.ds(wid * chunks_per_worker, chunks_per_worker)],
                    idx_v)
    pltpu.sync_copy(pos2_hbm, pos_v)

    pltpu.async_copy(table_hbm.at[idx_v.at[0]], rows_v.at[0], sem0)
    pltpu.async_copy(table_hbm.at[idx_v.at[1]], rows_v.at[1], sem1)

    out_base = wid * chunks_per_worker

    def outer(i, carry):
        for b in range(2):
            g = 2 * i + b
            sem = sem0 if b == 0 else sem1
            pltpu.make_async_copy(table_hbm.at[idx_v.at[g]], rows_v.at[b],
                                  sem).wait()
            p0 = lax.rem(g * CHUNK, MAXLEN)

            def add_row(r, carry2):
                for j in range(EMBED_DIM // 16):
                    sl = pl.ds(j * 16, 16)
                    plsc.addupdate(rows_v.at[b, r, sl], pos_v[p0 + r, sl])
                return carry2

            lax.fori_loop(0, CHUNK, add_row, 0, unroll=4)

            pltpu.sync_copy(rows_v.at[b],
                            out_hbm.at[pl.ds((out_base + g) * CHUNK, CHUNK)])

            @pl.when(g + 2 < chunks_per_worker)
            def _():
                pltpu.async_copy(table_hbm.at[idx_v.at[g + 2]], rows_v.at[b],
                                 sem)
        return carry

    lax.fori_loop(0, chunks_per_worker // 2, outer, 0)


def kernel(x, token_table, pos_table):
    batch, maxlen = x.shape
    embed_dim = token_table.shape[1]
    n_tokens = batch * maxlen
    n_chunks = n_tokens // CHUNK

    x_rows = x.reshape(n_chunks, CHUNK).astype(jnp.int32)
    pos2 = jnp.concatenate([pos_table, pos_table], axis=0)

    mesh = plsc.VectorSubcoreMesh(core_axis_name="c", subcore_axis_name="s")
    run = pl.kernel(
        _sc_body,
        out_type=jax.ShapeDtypeStruct((n_tokens, embed_dim), jnp.float32),
        mesh=mesh,
        compiler_params=pltpu.CompilerParams(use_tc_tiling_on_sc=False),
        scratch_types=[
            pltpu.VMEM((n_chunks // NUM_WORKERS, CHUNK), jnp.int32),
            pltpu.VMEM((2 * MAXLEN, embed_dim), jnp.float32),
            pltpu.VMEM((2, CHUNK, embed_dim), jnp.float32),
            pltpu.SemaphoreType.DMA,
            pltpu.SemaphoreType.DMA,
        ],
    )
    out = run(x_rows, pos2, token_table)
    return out.reshape(batch, maxlen, embed_dim)
